# baseline (speedup 1.0000x reference)
import functools

import jax
import jax.numpy as jnp
from jax import lax
from jax.experimental import pallas as pl
from jax.experimental.pallas import tpu as pltpu
from jax.experimental.pallas import tpu_sc as plsc

D_HID = 128
N_POS = 365
CHUNK = 32
NBUF = 8


@functools.partial(jax.jit, static_argnames=("batch", "seq"))
def _gather_rows(idx_flat, table, batch, seq):
    info = plsc.get_sparse_core_info()
    nc, ns = info.num_cores, info.num_subcores
    nw = nc * ns
    b_total = batch * seq
    b_per_w = b_total // nw
    n_chunks = b_per_w // CHUNK
    mesh = plsc.VectorSubcoreMesh(core_axis_name="c", subcore_axis_name="s")

    @functools.partial(
        pl.kernel,
        mesh=mesh,
        out_type=jax.ShapeDtypeStruct((b_total, D_HID), jnp.float32),
        scratch_types=(
            [pltpu.VMEM((b_per_w,), jnp.int32)]
            + [pltpu.VMEM((CHUNK, D_HID), jnp.float32) for _ in range(NBUF)]
            + [pltpu.VMEM_SHARED((N_POS, D_HID), jnp.float32)]
            + [pltpu.SemaphoreType.DMA for _ in range(2 * NBUF)]
        ),
    )
    def sc_kernel(idx_hbm, table_hbm, out_hbm, idx_v, *rest):
        rows = rest[:NBUF]
        table_sp = rest[NBUF]
        gsems = rest[NBUF + 1:2 * NBUF + 1]
        ssems = rest[2 * NBUF + 1:]

        wid = lax.axis_index("s") * nc + lax.axis_index("c")
        base = wid * b_per_w

        @pl.when(lax.axis_index("s") == 0)
        def _():
            pltpu.sync_copy(table_hbm, table_sp)

        plsc.subcore_barrier()
        pltpu.sync_copy(idx_hbm.at[pl.ds(base, b_per_w)], idx_v)

        def gather(c, b):
            off = pl.multiple_of(c * CHUNK, 8)
            pltpu.async_copy(
                table_sp.at[idx_v.at[pl.ds(off, CHUNK)]], rows[b], gsems[b]
            )

        def wait_gather(b):
            pltpu.make_async_copy(
                out_hbm.at[pl.ds(0, CHUNK)], rows[b], gsems[b]
            ).wait()

        def scatter(c, b):
            off = pl.multiple_of(base + c * CHUNK, 8)
            pltpu.async_copy(rows[b], out_hbm.at[pl.ds(off, CHUNK)], ssems[b])

        def wait_scatter(b):
            pltpu.make_async_copy(
                rows[b], out_hbm.at[pl.ds(0, CHUNK)], ssems[b]
            ).wait()

        AHEAD = 2
        for b in range(AHEAD):
            gather(b, b)

        def body(i, carry):
            for b in range(NBUF):
                c = NBUF * i + b
                wait_gather(b)

                nb = (b + AHEAD) % NBUF

                @pl.when(c + AHEAD < n_chunks)
                def _():
                    @pl.when(c >= NBUF - AHEAD)
                    def _():
                        wait_scatter(nb)

                    gather(c + AHEAD, nb)

                scatter(c, b)
            return carry

        lax.fori_loop(0, n_chunks // NBUF, body, 0)

        for b in range(NBUF):
            wait_scatter(b)

    return sc_kernel(idx_flat, table)


def kernel(doys, pos_table):
    batch, seq = doys.shape
    idx_flat = doys.astype(jnp.int32).reshape(batch * seq)
    out = _gather_rows(idx_flat, pos_table, batch, seq)
    return out.reshape(batch, seq, D_HID)


# speedup vs baseline: 1.0396x; 1.0396x over previous
import functools

import jax
import jax.numpy as jnp
from jax import lax
from jax.experimental import pallas as pl
from jax.experimental.pallas import tpu as pltpu
from jax.experimental.pallas import tpu_sc as plsc

D_HID = 128
N_POS = 365
CHUNK = 64
NBUF = 10


@functools.partial(jax.jit, static_argnames=("batch", "seq"))
def _gather_rows(idx_flat, table, batch, seq):
    info = plsc.get_sparse_core_info()
    nc, ns = info.num_cores, info.num_subcores
    nw = nc * ns
    b_total = batch * seq
    b_per_w = b_total // nw
    n_chunks = b_per_w // CHUNK
    mesh = plsc.VectorSubcoreMesh(core_axis_name="c", subcore_axis_name="s")

    @functools.partial(
        pl.kernel,
        mesh=mesh,
        out_type=jax.ShapeDtypeStruct((b_total, D_HID), jnp.float32),
        scratch_types=(
            [pltpu.VMEM((b_per_w,), jnp.int32)]
            + [pltpu.VMEM((CHUNK, D_HID), jnp.float32) for _ in range(NBUF)]
            + [pltpu.VMEM_SHARED((N_POS, D_HID), jnp.float32)]
            + [pltpu.SemaphoreType.DMA for _ in range(2 * NBUF)]
        ),
    )
    def sc_kernel(idx_hbm, table_hbm, out_hbm, idx_v, *rest):
        rows = rest[:NBUF]
        table_sp = rest[NBUF]
        gsems = rest[NBUF + 1:2 * NBUF + 1]
        ssems = rest[2 * NBUF + 1:]

        wid = lax.axis_index("s") * nc + lax.axis_index("c")
        base = wid * b_per_w

        @pl.when(lax.axis_index("s") == 0)
        def _():
            pltpu.sync_copy(table_hbm, table_sp)

        plsc.subcore_barrier()
        pltpu.sync_copy(idx_hbm.at[pl.ds(base, b_per_w)], idx_v)

        def gather(c, b):
            off = pl.multiple_of(c * CHUNK, 8)
            pltpu.async_copy(
                table_sp.at[idx_v.at[pl.ds(off, CHUNK)]], rows[b], gsems[b]
            )

        def wait_gather(b):
            pltpu.make_async_copy(
                out_hbm.at[pl.ds(0, CHUNK)], rows[b], gsems[b]
            ).wait()

        def scatter(c, b):
            off = pl.multiple_of(base + c * CHUNK, 8)
            pltpu.async_copy(rows[b], out_hbm.at[pl.ds(off, CHUNK)], ssems[b])

        def wait_scatter(b):
            pltpu.make_async_copy(
                rows[b], out_hbm.at[pl.ds(0, CHUNK)], ssems[b]
            ).wait()

        AHEAD = 2
        for b in range(AHEAD):
            gather(b, b)

        def body(i, carry):
            for b in range(NBUF):
                c = NBUF * i + b
                wait_gather(b)

                nb = (b + AHEAD) % NBUF

                @pl.when(c + AHEAD < n_chunks)
                def _():
                    @pl.when(c >= NBUF - AHEAD)
                    def _():
                        wait_scatter(nb)

                    gather(c + AHEAD, nb)

                scatter(c, b)
            return carry

        lax.fori_loop(0, n_chunks // NBUF, body, 0)

        for b in range(NBUF):
            wait_scatter(b)

    return sc_kernel(idx_flat, table)


def kernel(doys, pos_table):
    batch, seq = doys.shape
    idx_flat = doys.astype(jnp.int32).reshape(batch * seq)
    out = _gather_rows(idx_flat, pos_table, batch, seq)
    return out.reshape(batch, seq, D_HID)


# speedup vs baseline: 1.0444x; 1.0046x over previous
import functools

import jax
import jax.numpy as jnp
from jax import lax
from jax.experimental import pallas as pl
from jax.experimental.pallas import tpu as pltpu
from jax.experimental.pallas import tpu_sc as plsc

D_HID = 128
N_POS = 365
CHUNK = 64
NBUF = 8


@functools.partial(jax.jit, static_argnames=("batch", "seq"))
def _gather_rows(idx_flat, table, batch, seq):
    info = plsc.get_sparse_core_info()
    nc, ns = info.num_cores, info.num_subcores
    nw = nc * ns
    b_total = batch * seq
    b_per_w = b_total // nw
    n_chunks = b_per_w // CHUNK
    mesh = plsc.VectorSubcoreMesh(core_axis_name="c", subcore_axis_name="s")

    @functools.partial(
        pl.kernel,
        mesh=mesh,
        out_type=jax.ShapeDtypeStruct((b_total, D_HID), jnp.float32),
        scratch_types=(
            [pltpu.VMEM((b_per_w,), jnp.int32)]
            + [pltpu.VMEM((CHUNK, D_HID), jnp.float32) for _ in range(NBUF)]
            + [pltpu.VMEM_SHARED((N_POS, D_HID), jnp.float32)]
            + [pltpu.SemaphoreType.DMA for _ in range(2 * NBUF)]
        ),
    )
    def sc_kernel(idx_hbm, table_hbm, out_hbm, idx_v, *rest):
        rows = rest[:NBUF]
        table_sp = rest[NBUF]
        gsems = rest[NBUF + 1:2 * NBUF + 1]
        ssems = rest[2 * NBUF + 1:]

        wid = lax.axis_index("s") * nc + lax.axis_index("c")
        base = wid * b_per_w

        @pl.when(lax.axis_index("s") == 0)
        def _():
            pltpu.sync_copy(table_hbm, table_sp)

        plsc.subcore_barrier()
        pltpu.sync_copy(idx_hbm.at[pl.ds(base, b_per_w)], idx_v)

        def gather(c, b):
            off = pl.multiple_of(c * CHUNK, 8)
            pltpu.async_copy(
                table_sp.at[idx_v.at[pl.ds(off, CHUNK)]], rows[b], gsems[b]
            )

        def wait_gather(b):
            pltpu.make_async_copy(
                out_hbm.at[pl.ds(0, CHUNK)], rows[b], gsems[b]
            ).wait()

        def scatter(c, b):
            off = pl.multiple_of(base + c * CHUNK, 8)
            pltpu.async_copy(rows[b], out_hbm.at[pl.ds(off, CHUNK)], ssems[b])

        def wait_scatter(b):
            pltpu.make_async_copy(
                rows[b], out_hbm.at[pl.ds(0, CHUNK)], ssems[b]
            ).wait()

        AHEAD = 4
        for b in range(AHEAD):
            gather(b, b)

        def body(i, carry):
            for b in range(NBUF):
                c = NBUF * i + b
                wait_gather(b)

                nb = (b + AHEAD) % NBUF

                @pl.when(c + AHEAD < n_chunks)
                def _():
                    @pl.when(c >= NBUF - AHEAD)
                    def _():
                        wait_scatter(nb)

                    gather(c + AHEAD, nb)

                scatter(c, b)
            return carry

        lax.fori_loop(0, n_chunks // NBUF, body, 0)

        for b in range(NBUF):
            wait_scatter(b)

    return sc_kernel(idx_flat, table)


def kernel(doys, pos_table):
    batch, seq = doys.shape
    idx_flat = doys.astype(jnp.int32).reshape(batch * seq)
    out = _gather_rows(idx_flat, pos_table, batch, seq)
    return out.reshape(batch, seq, D_HID)


# speedup vs baseline: 1.0447x; 1.0003x over previous
import functools

import jax
import jax.numpy as jnp
from jax import lax
from jax.experimental import pallas as pl
from jax.experimental.pallas import tpu as pltpu
from jax.experimental.pallas import tpu_sc as plsc

D_HID = 128
N_POS = 365
CHUNK = 64
NBUF = 8


@functools.partial(jax.jit, static_argnames=("batch", "seq"))
def _gather_rows(idx_flat, table, batch, seq):
    info = plsc.get_sparse_core_info()
    nc, ns = info.num_cores, info.num_subcores
    nw = nc * ns
    b_total = batch * seq
    b_per_w = b_total // nw
    n_chunks = b_per_w // CHUNK
    mesh = plsc.VectorSubcoreMesh(core_axis_name="c", subcore_axis_name="s")

    @functools.partial(
        pl.kernel,
        mesh=mesh,
        out_type=jax.ShapeDtypeStruct((b_total, D_HID), jnp.float32),
        scratch_types=(
            [pltpu.VMEM((b_per_w,), jnp.int32)]
            + [pltpu.VMEM((CHUNK, D_HID), jnp.float32) for _ in range(NBUF)]
            + [pltpu.VMEM_SHARED((N_POS, D_HID), jnp.float32)]
            + [pltpu.SemaphoreType.DMA for _ in range(2 * NBUF)]
        ),
    )
    def sc_kernel(idx_hbm, table_hbm, out_hbm, idx_v, *rest):
        rows = rest[:NBUF]
        table_sp = rest[NBUF]
        gsems = rest[NBUF + 1:2 * NBUF + 1]
        ssems = rest[2 * NBUF + 1:]

        wid = lax.axis_index("s") * nc + lax.axis_index("c")
        base = wid * b_per_w

        @pl.when(lax.axis_index("s") == 0)
        def _():
            pltpu.sync_copy(table_hbm, table_sp)

        plsc.subcore_barrier()
        pltpu.sync_copy(idx_hbm.at[pl.ds(base, b_per_w)], idx_v)

        def gather(c, b):
            off = pl.multiple_of(c * CHUNK, 8)
            pltpu.async_copy(
                table_sp.at[idx_v.at[pl.ds(off, CHUNK)]], rows[b], gsems[b]
            )

        def wait_gather(b):
            pltpu.make_async_copy(
                out_hbm.at[pl.ds(0, CHUNK)], rows[b], gsems[b]
            ).wait()

        def scatter(c, b):
            off = pl.multiple_of(base + c * CHUNK, 8)
            pltpu.async_copy(rows[b], out_hbm.at[pl.ds(off, CHUNK)], ssems[b])

        def wait_scatter(b):
            pltpu.make_async_copy(
                rows[b], out_hbm.at[pl.ds(0, CHUNK)], ssems[b]
            ).wait()

        AHEAD = 6
        for b in range(AHEAD):
            gather(b, b)

        def body(i, carry):
            for b in range(NBUF):
                c = NBUF * i + b
                wait_gather(b)

                nb = (b + AHEAD) % NBUF

                @pl.when(c + AHEAD < n_chunks)
                def _():
                    @pl.when(c >= NBUF - AHEAD)
                    def _():
                        wait_scatter(nb)

                    gather(c + AHEAD, nb)

                scatter(c, b)
            return carry

        lax.fori_loop(0, n_chunks // NBUF, body, 0)

        for b in range(NBUF):
            wait_scatter(b)

    return sc_kernel(idx_flat, table)


def kernel(doys, pos_table):
    batch, seq = doys.shape
    idx_flat = doys.astype(jnp.int32).reshape(batch * seq)
    out = _gather_rows(idx_flat, pos_table, batch, seq)
    return out.reshape(batch, seq, D_HID)
